# baseline (speedup 1.0000x reference)
import jax
import jax.numpy as jnp
from jax import lax
from jax.experimental import pallas as pl
from jax.experimental.pallas import tpu as pltpu
from jax.experimental.pallas import tpu_sc as plsc

NUM_EMBEDDINGS = 1000000
EMBEDDING_DIM = 64
BATCH = 16384
HIST = 50

_INFO = plsc.get_sparse_core_info()
NC = _INFO.num_cores
NS = _INFO.num_subcores
NW = NC * NS

B_TOTAL = BATCH * HIST
B_PER_W = B_TOTAL // NW
CHUNK = 128
N_CHUNKS = B_PER_W // CHUNK
NBUF = 4
N_LOOP = N_CHUNKS // NBUF
TB_PER_H = BATCH // CHUNK
JT = EMBEDDING_DIM // 8

assert B_PER_W * NW == B_TOTAL
assert CHUNK * N_CHUNKS == B_PER_W
assert NBUF * N_LOOP == N_CHUNKS


def _body(table_hbm, idx_hbm, out_hbm, idx_v, *bufs):
  rows = bufs[:NBUF]
  tbufs = bufs[NBUF:2 * NBUF]
  gsems = bufs[2 * NBUF:3 * NBUF]
  wsems = bufs[3 * NBUF:]
  wid = lax.axis_index("s") * NC + lax.axis_index("c")
  chunk_base = wid * N_CHUNKS

  pltpu.sync_copy(idx_hbm.at[pl.ds(chunk_base, N_CHUNKS)], idx_v)

  lane = lax.iota(jnp.int32, 16)
  j_blocks = [(lane + (16 * i)) * CHUNK for i in range(EMBEDDING_DIM // 16)]

  def start_gather(g, b):
    pltpu.async_copy(table_hbm.at[idx_v.at[g]], rows[b], gsems[b])

  def wait_gather(g, b):
    pltpu.make_async_copy(table_hbm.at[idx_v.at[g]], rows[b], gsems[b]).wait()

  def for_each_write(g, b, fn):
    gg = chunk_base + g
    h = gg // TB_PER_H
    tb = gg % TB_PER_H
    for tj in range(JT):
      fn(tbufs[b].at[pl.ds(tj * 1024, 1024)], out_hbm.at[h, tj, tb])

  def start_write(g, b):
    for_each_write(g, b, lambda src, dst: pltpu.async_copy(src, dst, wsems[b]))

  def wait_write(g, b):
    for_each_write(
        g, b, lambda src, dst: pltpu.make_async_copy(src, dst, wsems[b]).wait()
    )

  def transpose_chunk(b):
    @pl.loop(0, CHUNK, step=8)
    def _(l0):
      for dl in range(8):
        l = l0 + dl
        for i in range(EMBEDDING_DIM // 16):
          v = rows[b][l, pl.ds(16 * i, 16)]
          plsc.store_scatter(tbufs[b], [j_blocks[i] + l], v)

  for b in range(NBUF):
    start_gather(b, b)

  @pl.loop(0, N_LOOP)
  def _(t):
    for b in range(NBUF):
      g = t * NBUF + b
      wait_gather(g, b)

      @pl.when(t > 0)
      def _():
        wait_write(g - NBUF, b)

      transpose_chunk(b)

      @pl.when(t < N_LOOP - 1)
      def _():
        start_gather(g + NBUF, b)

      start_write(g, b)

  for b in range(NBUF):
    wait_write(N_CHUNKS - NBUF + b, b)


@jax.jit
def kernel(input_, weight):
  idx = jnp.transpose(input_.astype(jnp.int32)).reshape(NW * N_CHUNKS, CHUNK)

  mesh = plsc.VectorSubcoreMesh(core_axis_name="c", subcore_axis_name="s")
  a4 = pl.kernel(
      _body,
      out_type=jax.ShapeDtypeStruct((HIST, JT, TB_PER_H, 8 * CHUNK), jnp.float32),
      mesh=mesh,
      compiler_params=pltpu.CompilerParams(
          use_tc_tiling_on_sc=False, needs_layout_passes=False
      ),
      scratch_types=(
          [pltpu.VMEM((N_CHUNKS, CHUNK), jnp.int32)]
          + [pltpu.VMEM((CHUNK, EMBEDDING_DIM), jnp.float32) for _ in range(NBUF)]
          + [pltpu.VMEM((EMBEDDING_DIM * CHUNK,), jnp.float32) for _ in range(NBUF)]
          + [pltpu.SemaphoreType.DMA for _ in range(2 * NBUF)]
      ),
  )(weight, idx)

  a5 = a4.reshape(HIST, JT, TB_PER_H, 8, CHUNK)
  return a5.transpose(2, 4, 0, 1, 3).reshape(BATCH, HIST, EMBEDDING_DIM)


# speedup vs baseline: 1.2194x; 1.2194x over previous
import jax
import jax.numpy as jnp
from jax import lax
from jax.experimental import pallas as pl
from jax.experimental.pallas import tpu as pltpu
from jax.experimental.pallas import tpu_sc as plsc

NUM_EMBEDDINGS = 1000000
EMBEDDING_DIM = 64
BATCH = 16384
HIST = 50

_INFO = plsc.get_sparse_core_info()
NC = _INFO.num_cores
NS = _INFO.num_subcores
NW = NC * NS

B_TOTAL = BATCH * HIST
B_PER_W = B_TOTAL // NW
CHUNK = 128
N_CHUNKS = B_PER_W // CHUNK
NBUF = 4
N_LOOP = N_CHUNKS // NBUF
TB_PER_H = BATCH // CHUNK
JT = EMBEDDING_DIM // 8

assert B_PER_W * NW == B_TOTAL
assert CHUNK * N_CHUNKS == B_PER_W
assert NBUF * N_LOOP == N_CHUNKS


def _body(table_hbm, idx_hbm, out_hbm, idx_v, *bufs):
  rows = bufs[:NBUF]
  tbufs = bufs[NBUF:2 * NBUF]
  gsems = bufs[2 * NBUF:3 * NBUF]
  wsems = bufs[3 * NBUF:]
  wid = lax.axis_index("s") * NC + lax.axis_index("c")
  chunk_base = wid * N_CHUNKS

  pltpu.sync_copy(idx_hbm.at[pl.ds(chunk_base, N_CHUNKS)], idx_v)

  lane = lax.iota(jnp.int32, 16)
  j_blocks = [(lane + (16 * i)) * CHUNK for i in range(EMBEDDING_DIM // 16)]

  def start_gather(g, b):
    pltpu.async_copy(table_hbm.at[idx_v.at[g]], rows[b], gsems[b])

  def wait_gather(g, b):
    pltpu.make_async_copy(table_hbm.at[idx_v.at[g]], rows[b], gsems[b]).wait()

  def for_each_write(g, b, fn):
    gg = chunk_base + g
    h = gg // TB_PER_H
    tb = gg % TB_PER_H
    for tj in range(JT):
      fn(tbufs[b].at[pl.ds(tj * 1024, 1024)], out_hbm.at[h, tj, tb])

  def start_write(g, b):
    for_each_write(g, b, lambda src, dst: pltpu.async_copy(src, dst, wsems[b]))

  def wait_write(g, b):
    for_each_write(
        g, b, lambda src, dst: pltpu.make_async_copy(src, dst, wsems[b]).wait()
    )

  def transpose_chunk(b):
    @plsc.parallel_loop(0, CHUNK, unroll=8)
    def _(l):
      for i in range(EMBEDDING_DIM // 16):
        v = rows[b][l, pl.ds(16 * i, 16)]
        plsc.store_scatter(tbufs[b], [j_blocks[i] + l], v)

  for b in range(NBUF):
    start_gather(b, b)

  @pl.loop(0, N_LOOP)
  def _(t):
    for b in range(NBUF):
      g = t * NBUF + b
      wait_gather(g, b)

      @pl.when(t > 0)
      def _():
        wait_write(g - NBUF, b)

      transpose_chunk(b)

      @pl.when(t < N_LOOP - 1)
      def _():
        start_gather(g + NBUF, b)

      start_write(g, b)

  for b in range(NBUF):
    wait_write(N_CHUNKS - NBUF + b, b)


@jax.jit
def kernel(input_, weight):
  idx = jnp.transpose(input_.astype(jnp.int32)).reshape(NW * N_CHUNKS, CHUNK)

  mesh = plsc.VectorSubcoreMesh(core_axis_name="c", subcore_axis_name="s")
  a4 = pl.kernel(
      _body,
      out_type=jax.ShapeDtypeStruct((HIST, JT, TB_PER_H, 8 * CHUNK), jnp.float32),
      mesh=mesh,
      compiler_params=pltpu.CompilerParams(
          use_tc_tiling_on_sc=False, needs_layout_passes=False
      ),
      scratch_types=(
          [pltpu.VMEM((N_CHUNKS, CHUNK), jnp.int32)]
          + [pltpu.VMEM((CHUNK, EMBEDDING_DIM), jnp.float32) for _ in range(NBUF)]
          + [pltpu.VMEM((EMBEDDING_DIM * CHUNK,), jnp.float32) for _ in range(NBUF)]
          + [pltpu.SemaphoreType.DMA for _ in range(2 * NBUF)]
      ),
  )(weight, idx)

  a5 = a4.reshape(HIST, JT, TB_PER_H, 8, CHUNK)
  return a5.transpose(2, 4, 0, 1, 3).reshape(BATCH, HIST, EMBEDDING_DIM)


# speedup vs baseline: 2.1223x; 1.7405x over previous
import jax
import jax.numpy as jnp
from jax import lax
from jax.experimental import pallas as pl
from jax.experimental.pallas import tpu as pltpu
from jax.experimental.pallas import tpu_sc as plsc

NUM_EMBEDDINGS = 1000000
EMBEDDING_DIM = 64
BATCH = 16384
HIST = 50

_INFO = plsc.get_sparse_core_info()
NC = _INFO.num_cores
NS = _INFO.num_subcores
NW = NC * NS

B_TOTAL = BATCH * HIST
B_PER_W = B_TOTAL // NW
CHUNK = 128
N_CHUNKS = B_PER_W // CHUNK
NBUF = 4
N_LOOP = N_CHUNKS // NBUF
TB_PER_H = BATCH // CHUNK
JT = EMBEDDING_DIM // 8
TPITCH = CHUNK + 1

assert B_PER_W * NW == B_TOTAL
assert CHUNK * N_CHUNKS == B_PER_W
assert NBUF * N_LOOP == N_CHUNKS


def _body(table_hbm, idx_hbm, out_hbm, idx_v, *bufs):
  rows = bufs[:NBUF]
  tbufs = bufs[NBUF:2 * NBUF]
  gsems = bufs[2 * NBUF:3 * NBUF]
  wsems = bufs[3 * NBUF:]
  wid = lax.axis_index("s") * NC + lax.axis_index("c")
  chunk_base = wid * N_CHUNKS

  pltpu.sync_copy(idx_hbm.at[pl.ds(chunk_base, N_CHUNKS)], idx_v)

  lane = lax.iota(jnp.int32, 16)
  j_blocks = [lane + (16 * i) for i in range(EMBEDDING_DIM // 16)]

  def start_gather(g, b):
    pltpu.async_copy(table_hbm.at[idx_v.at[g]], rows[b], gsems[b])

  def wait_gather(g, b):
    pltpu.make_async_copy(table_hbm.at[idx_v.at[g]], rows[b], gsems[b]).wait()

  def for_each_write(g, b, fn):
    gg = chunk_base + g
    h = gg // TB_PER_H
    tb = gg % TB_PER_H
    for tj in range(JT):
      fn(
          tbufs[b].at[pl.ds(tj * 8, 8), pl.ds(0, CHUNK)],
          out_hbm.at[h, tj, tb],
      )

  def start_write(g, b):
    for_each_write(g, b, lambda src, dst: pltpu.async_copy(src, dst, wsems[b]))

  def wait_write(g, b):
    for_each_write(
        g, b, lambda src, dst: pltpu.make_async_copy(src, dst, wsems[b]).wait()
    )

  def transpose_chunk(b):
    @plsc.parallel_loop(0, CHUNK, unroll=8)
    def _(l):
      l_vec = jnp.broadcast_to(l, (16,)).astype(jnp.int32)
      for i in range(EMBEDDING_DIM // 16):
        v = rows[b][l, pl.ds(16 * i, 16)]
        plsc.store_scatter(tbufs[b], [j_blocks[i], l_vec], v)

  for b in range(NBUF):
    start_gather(b, b)

  @pl.loop(0, N_LOOP)
  def _(t):
    for b in range(NBUF):
      g = t * NBUF + b
      wait_gather(g, b)

      @pl.when(t > 0)
      def _():
        wait_write(g - NBUF, b)

      transpose_chunk(b)

      @pl.when(t < N_LOOP - 1)
      def _():
        start_gather(g + NBUF, b)

      start_write(g, b)

  for b in range(NBUF):
    wait_write(N_CHUNKS - NBUF + b, b)


@jax.jit
def kernel(input_, weight):
  idx = jnp.transpose(input_.astype(jnp.int32)).reshape(NW * N_CHUNKS, CHUNK)

  mesh = plsc.VectorSubcoreMesh(core_axis_name="c", subcore_axis_name="s")
  a4 = pl.kernel(
      _body,
      out_type=jax.ShapeDtypeStruct((HIST, JT, TB_PER_H, 8, CHUNK), jnp.float32),
      mesh=mesh,
      compiler_params=pltpu.CompilerParams(
          use_tc_tiling_on_sc=False, needs_layout_passes=False
      ),
      scratch_types=(
          [pltpu.VMEM((N_CHUNKS, CHUNK), jnp.int32)]
          + [pltpu.VMEM((CHUNK, EMBEDDING_DIM), jnp.float32) for _ in range(NBUF)]
          + [pltpu.VMEM((EMBEDDING_DIM, TPITCH), jnp.float32) for _ in range(NBUF)]
          + [pltpu.SemaphoreType.DMA for _ in range(2 * NBUF)]
      ),
  )(weight, idx)

  return a4.transpose(2, 4, 0, 1, 3).reshape(BATCH, HIST, EMBEDDING_DIM)
